# baseline (speedup 1.0000x reference)
import jax
import jax.numpy as jnp
from jax import lax
from jax.experimental import pallas as pl
from jax.experimental.pallas import tpu as pltpu
from jax.experimental.pallas import tpu_sc as plsc

N = 10000
E = 320000
G = 128
STEPS = 4
CAT = 2 * G

NP_ = 10240
NW = 32
CHUNK = 128
EP = 327680
ROWS_PER_TILE = EP // NW // CHUNK
R_C0 = 48
R_C1 = 112
R_MAX = max(R_C0, R_C1)
EPK_ROWS = EP // CHUNK + R_MAX
SROWS = NP_ // 16
_IDX_GRP = 8

_F32 = jnp.float32


def _make_sc_step():
    mesh = plsc.VectorSubcoreMesh(core_axis_name="c", subcore_axis_name="s")
    out_type = jax.ShapeDtypeStruct((2 * NP_, G), _F32)
    scratch = [
        pltpu.VMEM((CHUNK, G), _F32),
        pltpu.VMEM((CHUNK, G), _F32),
        pltpu.VMEM((R_MAX, CHUNK), jnp.int32),
        pltpu.VMEM((2, CHUNK), jnp.int32),
        pltpu.VMEM((2, CHUNK), jnp.int32),
        pltpu.VMEM_SHARED((NP_, G), _F32),
        pltpu.SemaphoreType.DMA,
        pltpu.SemaphoreType.DMA,
        pltpu.SemaphoreType.DMA,
        pltpu.SemaphoreType.DMA,
        pltpu.SemaphoreType.DMA,
        pltpu.SemaphoreType.DMA,
    ]

    def body(p_hbm, q_hbm, pk_hbm, s_out,
             pb0, pb1, pk, dc, sc_, s_sh, gp0, gp1, gq0, gq1, sc0, sc1):
        pbs = (pb0, pb1)
        gps, gqs, scs = (gp0, gp1), (gq0, gq1), (sc0, sc1)
        c = lax.axis_index("c")
        s = lax.axis_index("s")
        wid = s * 2 + c
        zero16 = jnp.zeros((16,), _F32)

        nrows = jnp.where(c == 0, R_C0, R_C1)
        base = jnp.where(c == 0, s * R_C0, 16 * R_C0 + s * R_C1)
        pltpu.sync_copy(pk_hbm.at[pl.ds(base, R_MAX)], pk)

        def zrow(r, _):
            for cc in range(G // 16):
                pb0[r, pl.ds(cc * 16, 16)] = zero16
            return 0
        lax.fori_loop(0, CHUNK, zrow, 0)
        for k in range(SROWS // CHUNK):
            pltpu.sync_copy(pb0, s_sh.at[pl.ds(s * SROWS + k * CHUNK, CHUNK)])
        plsc.subcore_barrier()

        def unpack(j, b):
            for cc in range(G // 16):
                sl = pl.ds(cc * 16, 16)
                v = pk[j, sl]
                dc[b, sl] = jnp.bitwise_and(v, 0xFFFF)
                sc_[b, sl] = lax.shift_right_logical(v, 16)

        def start_p(b):
            pltpu.async_copy(p_hbm.at[dc.at[b]], pbs[b], gps[b])

        def wait_p(b):
            pltpu.make_async_copy(p_hbm.at[dc.at[b]], pbs[b], gps[b]).wait()

        def start_q_add(b):
            pltpu.async_copy(q_hbm.at[sc_.at[b]], pbs[b], gqs[b], add=True)

        def wait_q(b):
            pltpu.make_async_copy(q_hbm.at[sc_.at[b]], pbs[b], gqs[b]).wait()

        def start_scatter(b):
            pltpu.async_copy(pbs[b], s_sh.at[dc.at[b]], scs[b], add=True)

        def wait_scatter(b):
            pltpu.make_async_copy(pbs[b], s_sh.at[dc.at[b]], scs[b]).wait()

        def compute(b):
            pb = pbs[b]

            @plsc.parallel_loop(0, CHUNK, unroll=4)
            def trow(r):
                for cc in range(G // 16):
                    sl = pl.ds(cc * 16, 16)
                    v = pb[r, sl]
                    pb[r, sl] = 2.0 / (1.0 + jnp.exp(v * -2.0)) - 1.0

        unpack(0, 0)
        start_p(0)

        def grp(g, _):
            for b in range(2):
                j = 2 * g + b
                nb = 1 - b
                wait_p(b)
                start_q_add(b)
                if b == 0:
                    @pl.when(g > 0)
                    def _():
                        wait_scatter(nb)
                    unpack(j + 1, nb)
                    start_p(nb)
                else:
                    wait_scatter(nb)

                    @pl.when(g < nrows // 2 - 1)
                    def _():
                        unpack(j + 1, nb)
                        start_p(nb)
                wait_q(b)
                compute(b)
                start_scatter(b)
            return 0
        lax.fori_loop(0, nrows // 2, grp, 0)
        wait_scatter(1)
        plsc.subcore_barrier()

        for k in range(SROWS // CHUNK):
            rows = pl.ds(s * SROWS + k * CHUNK, CHUNK)
            pltpu.sync_copy(s_sh.at[rows], pb0)
            pltpu.sync_copy(pb0, s_out.at[pl.ds(c * NP_ + s * SROWS + k * CHUNK, CHUNK)])

    return pl.kernel(body, out_type=out_type, mesh=mesh,
                     scratch_types=scratch, name="sc_edge_step")


def _make_sc_deg():
    mesh = plsc.VectorSubcoreMesh(core_axis_name="c", subcore_axis_name="s")
    out_type = jax.ShapeDtypeStruct((2 * NP_, G), _F32)
    scratch = [
        pltpu.VMEM((CHUNK, G), _F32),
        pltpu.VMEM((_IDX_GRP, CHUNK), jnp.int32),
        pltpu.VMEM_SHARED((NP_, G), _F32),
    ]

    def body(dst_hbm, d_out, buf, dstv, deg_sh):
        c = lax.axis_index("c")
        s = lax.axis_index("s")
        wid = s * 2 + c
        zero16 = jnp.zeros((16,), _F32)
        one16 = jnp.ones((16,), _F32)

        def zrow(r, _):
            for cc in range(G // 16):
                buf[r, pl.ds(cc * 16, 16)] = zero16
            return 0
        lax.fori_loop(0, CHUNK, zrow, 0)
        for k in range(SROWS // CHUNK):
            pltpu.sync_copy(buf, deg_sh.at[pl.ds(s * SROWS + k * CHUNK, CHUNK)])

        def orow(r, _):
            buf[r, pl.ds(0, 16)] = one16
            return 0
        lax.fori_loop(0, CHUNK, orow, 0)
        plsc.subcore_barrier()

        def grp_body(g, _):
            base = wid * ROWS_PER_TILE + g * _IDX_GRP
            pltpu.sync_copy(dst_hbm.at[pl.ds(base, _IDX_GRP)], dstv)

            def chunk_body(j, _):
                pltpu.sync_copy(buf, deg_sh.at[dstv.at[j]], add=True)
                return 0
            lax.fori_loop(0, _IDX_GRP, chunk_body, 0)
            return 0
        lax.fori_loop(0, ROWS_PER_TILE // _IDX_GRP, grp_body, 0)
        plsc.subcore_barrier()

        for k in range(SROWS // CHUNK):
            rows = pl.ds(s * SROWS + k * CHUNK, CHUNK)
            pltpu.sync_copy(deg_sh.at[rows], buf)
            pltpu.sync_copy(buf, d_out.at[pl.ds(c * NP_ + s * SROWS + k * CHUNK, CHUNK)])

    return pl.kernel(body, out_type=out_type, mesh=mesh,
                     scratch_types=scratch, name="sc_deg")


_sc_step = _make_sc_step()
_sc_deg = _make_sc_deg()


_BLK = 128
_GRID = NP_ // _BLK

_full = lambda shape: pl.BlockSpec(shape, lambda i: (0,) * len(shape))
_rows = lambda shape: pl.BlockSpec(shape, lambda i: (i, 0))
_core = lambda cc, shape: pl.BlockSpec(shape, lambda i, _c=cc: (_c * _GRID + i, 0))


def _pre_body(h_ref, a_ref, b_ref, b1_ref, p_ref, q_ref):
    h = h_ref[...]
    p_ref[...] = jnp.dot(h, a_ref[...], preferred_element_type=_F32, precision=lax.Precision.HIGHEST) + b1_ref[...]
    q_ref[...] = jnp.dot(h, b_ref[...], preferred_element_type=_F32, precision=lax.Precision.HIGHEST)


_tc_pre = pl.pallas_call(
    _pre_body,
    grid=(_GRID,),
    in_specs=[_rows((_BLK, G)), _full((G, G)), _full((G, G)), _full((1, G))],
    out_specs=[_rows((_BLK, G)), _rows((_BLK, G))],
    out_shape=[jax.ShapeDtypeStruct((NP_, G), _F32),
               jax.ShapeDtypeStruct((NP_, G), _F32)],
)


def _combine_h(s0_ref, s1_ref, d0_ref, d1_ref, w2_ref, b2_ref):
    deg = (d0_ref[...] + d1_ref[...])[:, 0:1]
    ssum = s0_ref[...] + s1_ref[...]
    return (jnp.dot(ssum, w2_ref[...], preferred_element_type=_F32, precision=lax.Precision.HIGHEST)
            + deg * b2_ref[...])


def _mid_body(s0_ref, s1_ref, d0_ref, d1_ref, w2_ref, b2_ref, a_ref, b_ref,
              b1_ref, p_ref, q_ref):
    h = _combine_h(s0_ref, s1_ref, d0_ref, d1_ref, w2_ref, b2_ref)
    p_ref[...] = jnp.dot(h, a_ref[...], preferred_element_type=_F32, precision=lax.Precision.HIGHEST) + b1_ref[...]
    q_ref[...] = jnp.dot(h, b_ref[...], preferred_element_type=_F32, precision=lax.Precision.HIGHEST)


_tc_mid = pl.pallas_call(
    _mid_body,
    grid=(_GRID,),
    in_specs=[_core(0, (_BLK, G)), _core(1, (_BLK, G)),
              _core(0, (_BLK, G)), _core(1, (_BLK, G)),
              _full((G, G)), _full((1, G)),
              _full((G, G)), _full((G, G)), _full((1, G))],
    out_specs=[_rows((_BLK, G)), _rows((_BLK, G))],
    out_shape=[jax.ShapeDtypeStruct((NP_, G), _F32),
               jax.ShapeDtypeStruct((NP_, G), _F32)],
)


def _final_body(s0_ref, s1_ref, d0_ref, d1_ref, w2_ref, b2_ref, x_ref,
                g_ref, b_ref, ow_ref, ob_ref, o_ref):
    h = _combine_h(s0_ref, s1_ref, d0_ref, d1_ref, w2_ref, b2_ref)
    cat = jnp.concatenate([x_ref[...], h], axis=-1)
    mu = jnp.mean(cat, axis=-1, keepdims=True)
    var = jnp.mean((cat - mu) ** 2, axis=-1, keepdims=True)
    normed = (cat - mu) * lax.rsqrt(var + 1e-5) * g_ref[...] + b_ref[...]
    o_ref[...] = jnp.dot(normed, ow_ref[...], preferred_element_type=_F32, precision=lax.Precision.HIGHEST) + ob_ref[...]


_tc_final = pl.pallas_call(
    _final_body,
    grid=(_GRID,),
    in_specs=[_core(0, (_BLK, G)), _core(1, (_BLK, G)),
              _core(0, (_BLK, G)), _core(1, (_BLK, G)),
              _full((G, G)), _full((1, G)),
              _rows((_BLK, G)),
              _full((1, CAT)), _full((1, CAT)),
              _full((CAT, _BLK)), _full((1, _BLK))],
    out_specs=[_rows((_BLK, _BLK))],
    out_shape=[jax.ShapeDtypeStruct((NP_, _BLK), _F32)],
)


def kernel(x, edge_index, gnn_W1, gnn_b1, gnn_W2, gnn_b2, ln_g, ln_b, out_W, out_b):
    xp = jnp.zeros((NP_, G), _F32).at[:N].set(x)
    dst = edge_index[1]
    src = edge_index[0]
    dstp = jnp.full((EP,), N, jnp.int32).at[:E].set(dst).reshape(EP // CHUNK, CHUNK)
    srcp = jnp.zeros((EP,), jnp.int32).at[:E].set(src).reshape(EP // CHUNK, CHUNK)
    w1a = gnn_W1[:, :G, :].astype(jnp.bfloat16).astype(_F32)
    w1b = gnn_W1[:, G:, :].astype(jnp.bfloat16).astype(_F32)
    w2q = gnn_W2.astype(jnp.bfloat16).astype(_F32)
    A = w1a - w1b
    B = w1b
    b1 = gnn_b1.reshape(STEPS, 1, G)
    b2 = gnn_b2.reshape(STEPS, 1, G)
    owp = jnp.zeros((CAT, _BLK), _F32).at[:, :3].set(out_W)
    obp = jnp.zeros((1, _BLK), _F32).at[0, :3].set(out_b)

    pk = jnp.bitwise_or(dstp, jnp.left_shift(srcp, 16))
    pk = jnp.full((EPK_ROWS, CHUNK), N, jnp.int32).at[:EP // CHUNK].set(pk)

    d = _sc_deg(dstp)
    p, q = _tc_pre(xp, A[0], B[0], b1[0])
    s = _sc_step(p, q, pk)
    for i in range(1, STEPS):
        p, q = _tc_mid(s, s, d, d, w2q[i - 1], b2[i - 1], A[i], B[i], b1[i])
        s = _sc_step(p, q, pk)
    (outp,) = _tc_final(s, s, d, d, w2q[STEPS - 1], b2[STEPS - 1], xp,
                        ln_g.reshape(1, CAT), ln_b.reshape(1, CAT), owp, obp)
    return outp[:N, :3]


# speedup vs baseline: 1.2659x; 1.2659x over previous
import jax
import jax.numpy as jnp
from jax import lax
from jax.experimental import pallas as pl
from jax.experimental.pallas import tpu as pltpu
from jax.experimental.pallas import tpu_sc as plsc

N = 10000
E = 320000
G = 128
STEPS = 4
CAT = 2 * G

NP_ = 10240
NW = 32
CHUNK = 128
EP = 327680
ROWS_PER_TILE = EP // NW // CHUNK
R_C0 = 112
R_C1 = 48
R_MAX = max(R_C0, R_C1)
EPK_ROWS = EP // CHUNK + R_MAX
SROWS = NP_ // 16
_IDX_GRP = 8

_F32 = jnp.float32


def _make_sc_step():
    mesh = plsc.VectorSubcoreMesh(core_axis_name="c", subcore_axis_name="s")
    out_type = jax.ShapeDtypeStruct((2 * NP_, G), _F32)
    scratch = [
        pltpu.VMEM((CHUNK, G), _F32),
        pltpu.VMEM((CHUNK, G), _F32),
        pltpu.VMEM((R_MAX, CHUNK), jnp.int32),
        pltpu.VMEM((2, CHUNK), jnp.int32),
        pltpu.VMEM((2, CHUNK), jnp.int32),
        pltpu.VMEM_SHARED((NP_, G), _F32),
        pltpu.SemaphoreType.DMA,
        pltpu.SemaphoreType.DMA,
        pltpu.SemaphoreType.DMA,
        pltpu.SemaphoreType.DMA,
        pltpu.SemaphoreType.DMA,
        pltpu.SemaphoreType.DMA,
    ]

    def body(p_hbm, q_hbm, pk_hbm, s_out,
             pb0, pb1, pk, dc, sc_, s_sh, gp0, gp1, gq0, gq1, sc0, sc1):
        pbs = (pb0, pb1)
        gps, gqs, scs = (gp0, gp1), (gq0, gq1), (sc0, sc1)
        c = lax.axis_index("c")
        s = lax.axis_index("s")
        wid = s * 2 + c
        zero16 = jnp.zeros((16,), _F32)

        nrows = jnp.where(c == 0, R_C0, R_C1)
        base = jnp.where(c == 0, s * R_C0, 16 * R_C0 + s * R_C1)
        pltpu.sync_copy(pk_hbm.at[pl.ds(base, R_MAX)], pk)

        def zrow(r, _):
            for cc in range(G // 16):
                pb0[r, pl.ds(cc * 16, 16)] = zero16
            return 0
        lax.fori_loop(0, CHUNK, zrow, 0)
        for k in range(SROWS // CHUNK):
            pltpu.sync_copy(pb0, s_sh.at[pl.ds(s * SROWS + k * CHUNK, CHUNK)])
        plsc.subcore_barrier()

        def unpack(j, b):
            for cc in range(G // 16):
                sl = pl.ds(cc * 16, 16)
                v = pk[j, sl]
                dc[b, sl] = jnp.bitwise_and(v, 0xFFFF)
                sc_[b, sl] = lax.shift_right_logical(v, 16)

        def start_p(b):
            pltpu.async_copy(p_hbm.at[dc.at[b]], pbs[b], gps[b])

        def wait_p(b):
            pltpu.make_async_copy(p_hbm.at[dc.at[b]], pbs[b], gps[b]).wait()

        def start_q_add(b):
            pltpu.async_copy(q_hbm.at[sc_.at[b]], pbs[b], gqs[b], add=True)

        def wait_q(b):
            pltpu.make_async_copy(q_hbm.at[sc_.at[b]], pbs[b], gqs[b]).wait()

        def start_scatter(b):
            pltpu.async_copy(pbs[b], s_sh.at[dc.at[b]], scs[b], add=True)

        def wait_scatter(b):
            pltpu.make_async_copy(pbs[b], s_sh.at[dc.at[b]], scs[b]).wait()

        def compute(b):
            pb = pbs[b]

            @plsc.parallel_loop(0, CHUNK, unroll=4)
            def trow(r):
                for cc in range(G // 16):
                    sl = pl.ds(cc * 16, 16)
                    v = pb[r, sl]
                    pb[r, sl] = 2.0 / (1.0 + jnp.exp(v * -2.0)) - 1.0

        unpack(0, 0)
        start_p(0)

        def grp(g, _):
            for b in range(2):
                j = 2 * g + b
                nb = 1 - b
                wait_p(b)
                start_q_add(b)
                if b == 0:
                    @pl.when(g > 0)
                    def _():
                        wait_scatter(nb)
                    unpack(j + 1, nb)
                    start_p(nb)
                else:
                    wait_scatter(nb)

                    @pl.when(g < nrows // 2 - 1)
                    def _():
                        unpack(j + 1, nb)
                        start_p(nb)
                wait_q(b)
                compute(b)
                start_scatter(b)
            return 0
        lax.fori_loop(0, nrows // 2, grp, 0)
        wait_scatter(1)
        plsc.subcore_barrier()

        for k in range(SROWS // CHUNK):
            rows = pl.ds(s * SROWS + k * CHUNK, CHUNK)
            pltpu.sync_copy(s_sh.at[rows], pb0)
            pltpu.sync_copy(pb0, s_out.at[pl.ds(c * NP_ + s * SROWS + k * CHUNK, CHUNK)])

    return pl.kernel(body, out_type=out_type, mesh=mesh,
                     scratch_types=scratch, name="sc_edge_step")


def _make_sc_deg():
    mesh = plsc.VectorSubcoreMesh(core_axis_name="c", subcore_axis_name="s")
    out_type = jax.ShapeDtypeStruct((2 * NP_, G), _F32)
    scratch = [
        pltpu.VMEM((CHUNK, G), _F32),
        pltpu.VMEM((_IDX_GRP, CHUNK), jnp.int32),
        pltpu.VMEM_SHARED((NP_, G), _F32),
    ]

    def body(dst_hbm, d_out, buf, dstv, deg_sh):
        c = lax.axis_index("c")
        s = lax.axis_index("s")
        wid = s * 2 + c
        zero16 = jnp.zeros((16,), _F32)
        one16 = jnp.ones((16,), _F32)

        def zrow(r, _):
            for cc in range(G // 16):
                buf[r, pl.ds(cc * 16, 16)] = zero16
            return 0
        lax.fori_loop(0, CHUNK, zrow, 0)
        for k in range(SROWS // CHUNK):
            pltpu.sync_copy(buf, deg_sh.at[pl.ds(s * SROWS + k * CHUNK, CHUNK)])

        def orow(r, _):
            buf[r, pl.ds(0, 16)] = one16
            return 0
        lax.fori_loop(0, CHUNK, orow, 0)
        plsc.subcore_barrier()

        def grp_body(g, _):
            base = wid * ROWS_PER_TILE + g * _IDX_GRP
            pltpu.sync_copy(dst_hbm.at[pl.ds(base, _IDX_GRP)], dstv)

            def chunk_body(j, _):
                pltpu.sync_copy(buf, deg_sh.at[dstv.at[j]], add=True)
                return 0
            lax.fori_loop(0, _IDX_GRP, chunk_body, 0)
            return 0
        lax.fori_loop(0, ROWS_PER_TILE // _IDX_GRP, grp_body, 0)
        plsc.subcore_barrier()

        for k in range(SROWS // CHUNK):
            rows = pl.ds(s * SROWS + k * CHUNK, CHUNK)
            pltpu.sync_copy(deg_sh.at[rows], buf)
            pltpu.sync_copy(buf, d_out.at[pl.ds(c * NP_ + s * SROWS + k * CHUNK, CHUNK)])

    return pl.kernel(body, out_type=out_type, mesh=mesh,
                     scratch_types=scratch, name="sc_deg")


_sc_step = _make_sc_step()
_sc_deg = _make_sc_deg()


_BLK = 128
_GRID = NP_ // _BLK

_full = lambda shape: pl.BlockSpec(shape, lambda i: (0,) * len(shape))
_rows = lambda shape: pl.BlockSpec(shape, lambda i: (i, 0))
_core = lambda cc, shape: pl.BlockSpec(shape, lambda i, _c=cc: (_c * _GRID + i, 0))


def _pre_body(h_ref, a_ref, b_ref, b1_ref, p_ref, q_ref):
    h = h_ref[...]
    p_ref[...] = jnp.dot(h, a_ref[...], preferred_element_type=_F32, precision=lax.Precision.HIGHEST) + b1_ref[...]
    q_ref[...] = jnp.dot(h, b_ref[...], preferred_element_type=_F32, precision=lax.Precision.HIGHEST)


_tc_pre = pl.pallas_call(
    _pre_body,
    grid=(_GRID,),
    in_specs=[_rows((_BLK, G)), _full((G, G)), _full((G, G)), _full((1, G))],
    out_specs=[_rows((_BLK, G)), _rows((_BLK, G))],
    out_shape=[jax.ShapeDtypeStruct((NP_, G), _F32),
               jax.ShapeDtypeStruct((NP_, G), _F32)],
)


def _combine_h(s0_ref, s1_ref, d0_ref, d1_ref, w2_ref, b2_ref):
    deg = (d0_ref[...] + d1_ref[...])[:, 0:1]
    ssum = s0_ref[...] + s1_ref[...]
    return (jnp.dot(ssum, w2_ref[...], preferred_element_type=_F32, precision=lax.Precision.HIGHEST)
            + deg * b2_ref[...])


def _mid_body(s0_ref, s1_ref, d0_ref, d1_ref, w2_ref, b2_ref, a_ref, b_ref,
              b1_ref, p_ref, q_ref):
    h = _combine_h(s0_ref, s1_ref, d0_ref, d1_ref, w2_ref, b2_ref)
    p_ref[...] = jnp.dot(h, a_ref[...], preferred_element_type=_F32, precision=lax.Precision.HIGHEST) + b1_ref[...]
    q_ref[...] = jnp.dot(h, b_ref[...], preferred_element_type=_F32, precision=lax.Precision.HIGHEST)


_tc_mid = pl.pallas_call(
    _mid_body,
    grid=(_GRID,),
    in_specs=[_core(0, (_BLK, G)), _core(1, (_BLK, G)),
              _core(0, (_BLK, G)), _core(1, (_BLK, G)),
              _full((G, G)), _full((1, G)),
              _full((G, G)), _full((G, G)), _full((1, G))],
    out_specs=[_rows((_BLK, G)), _rows((_BLK, G))],
    out_shape=[jax.ShapeDtypeStruct((NP_, G), _F32),
               jax.ShapeDtypeStruct((NP_, G), _F32)],
)


def _final_body(s0_ref, s1_ref, d0_ref, d1_ref, w2_ref, b2_ref, x_ref,
                g_ref, b_ref, ow_ref, ob_ref, o_ref):
    h = _combine_h(s0_ref, s1_ref, d0_ref, d1_ref, w2_ref, b2_ref)
    cat = jnp.concatenate([x_ref[...], h], axis=-1)
    mu = jnp.mean(cat, axis=-1, keepdims=True)
    var = jnp.mean((cat - mu) ** 2, axis=-1, keepdims=True)
    normed = (cat - mu) * lax.rsqrt(var + 1e-5) * g_ref[...] + b_ref[...]
    o_ref[...] = jnp.dot(normed, ow_ref[...], preferred_element_type=_F32, precision=lax.Precision.HIGHEST) + ob_ref[...]


_tc_final = pl.pallas_call(
    _final_body,
    grid=(_GRID,),
    in_specs=[_core(0, (_BLK, G)), _core(1, (_BLK, G)),
              _core(0, (_BLK, G)), _core(1, (_BLK, G)),
              _full((G, G)), _full((1, G)),
              _rows((_BLK, G)),
              _full((1, CAT)), _full((1, CAT)),
              _full((CAT, _BLK)), _full((1, _BLK))],
    out_specs=[_rows((_BLK, _BLK))],
    out_shape=[jax.ShapeDtypeStruct((NP_, _BLK), _F32)],
)


def kernel(x, edge_index, gnn_W1, gnn_b1, gnn_W2, gnn_b2, ln_g, ln_b, out_W, out_b):
    xp = jnp.zeros((NP_, G), _F32).at[:N].set(x)
    dst = edge_index[1]
    src = edge_index[0]
    dstp = jnp.full((EP,), N, jnp.int32).at[:E].set(dst).reshape(EP // CHUNK, CHUNK)
    srcp = jnp.zeros((EP,), jnp.int32).at[:E].set(src).reshape(EP // CHUNK, CHUNK)
    w1a = gnn_W1[:, :G, :].astype(jnp.bfloat16).astype(_F32)
    w1b = gnn_W1[:, G:, :].astype(jnp.bfloat16).astype(_F32)
    w2q = gnn_W2.astype(jnp.bfloat16).astype(_F32)
    A = w1a - w1b
    B = w1b
    b1 = gnn_b1.reshape(STEPS, 1, G)
    b2 = gnn_b2.reshape(STEPS, 1, G)
    owp = jnp.zeros((CAT, _BLK), _F32).at[:, :3].set(out_W)
    obp = jnp.zeros((1, _BLK), _F32).at[0, :3].set(out_b)

    pk = jnp.bitwise_or(dstp, jnp.left_shift(srcp, 16))
    pk = jnp.full((EPK_ROWS, CHUNK), N, jnp.int32).at[:EP // CHUNK].set(pk)

    d = _sc_deg(dstp)
    p, q = _tc_pre(xp, A[0], B[0], b1[0])
    s = _sc_step(p, q, pk)
    for i in range(1, STEPS):
        p, q = _tc_mid(s, s, d, d, w2q[i - 1], b2[i - 1], A[i], B[i], b1[i])
        s = _sc_step(p, q, pk)
    (outp,) = _tc_final(s, s, d, d, w2q[STEPS - 1], b2[STEPS - 1], xp,
                        ln_g.reshape(1, CAT), ln_b.reshape(1, CAT), owp, obp)
    return outp[:N, :3]
